# Initial kernel scaffold; baseline (speedup 1.0000x reference)
import jax
import jax.numpy as jnp
from jax.experimental import pallas as pl


def kernel(vals, distances, Wk1, bk1, Wk2, bk2, Wl1, bl1, Wl2, bl2, Wt1, bt1, Wt2, bt2):
    raise NotImplementedError("write your pallas kernel here")


# speedup vs baseline: 1.7706x; 1.7706x over previous
import functools
import jax
import jax.numpy as jnp
from jax import lax
from jax.experimental import pallas as pl
from jax.experimental.pallas import tpu as pltpu
from jax.experimental.pallas import tpu_sc as plsc

_V = 100000
_R = 6
_NW = 32
_L = 16


def _dense_body(vals_ref, dist_ref, wk1_ref, bk1_ref, wk2_ref, bk2_ref,
                wt1_ref, bt1_ref, wt2_ref, bt2_ref, w2_ref):
    vals = vals_ref[...]
    dist = dist_ref[...]
    T, K = vals.shape

    eq = vals[:, :, None] == vals[:, None, :]
    ji = lax.broadcasted_iota(jnp.int32, (K, K), 0)
    mi = lax.broadcasted_iota(jnp.int32, (K, K), 1)
    seen = jnp.any(eq & (mi < ji)[None], axis=-1)
    isnew = ((vals != 0) & ~seen).astype(jnp.float32)
    cumtri = (mi <= ji).astype(jnp.float32)
    counts = lax.dot_general(isnew, cumtri, (((1,), (1,)), ((), ())),
                             preferred_element_type=jnp.float32)

    net_in = jnp.concatenate([dist, counts], axis=-1)

    h_k = jnp.tanh(jnp.dot(net_in, wk1_ref[...],
                           preferred_element_type=jnp.float32) + bk1_ref[...])
    logits = jnp.dot(h_k, wk2_ref[...],
                     preferred_element_type=jnp.float32) + bk2_ref[...]
    logits = logits - jnp.max(logits, axis=-1, keepdims=True)
    e = jnp.exp(logits)
    k_probs = e / jnp.sum(e, axis=-1, keepdims=True)

    h_t = jnp.tanh(jnp.dot(net_in, wt1_ref[...],
                           preferred_element_type=jnp.float32) + bt1_ref[...])
    tz = jnp.dot(h_t, wt2_ref[...],
                 preferred_element_type=jnp.float32) + bt2_ref[...]
    temp = 1.0 / (1.0 + jnp.exp(-tz))

    ri = lax.broadcasted_iota(jnp.int32, (_R, K), 0)
    ki = lax.broadcasted_iota(jnp.int32, (_R, K), 1)
    kmask = jnp.where(ki < (1 << ri), 1.0, 1000.0).astype(jnp.float32)

    d = -(dist[:, None, :] * kmask[None, :, :]) / temp[:, :, None]
    d = d - jnp.max(d, axis=-1, keepdims=True)
    ed = jnp.exp(d)
    kw = ed / jnp.sum(ed, axis=-1, keepdims=True)
    w = jnp.sum(k_probs[:, :, None] * kw, axis=1)

    w2_ref[...] = jnp.sum(eq.astype(jnp.float32) * w[:, None, :], axis=-1)


def _dense_pallas(vals_f, dist_f, wk1t, bk1, wk2t, bk2, wt1t, bt1, wt2t, bt2):
    T, K = vals_f.shape
    return pl.pallas_call(
        _dense_body,
        out_shape=jax.ShapeDtypeStruct((T, K), jnp.float32),
    )(vals_f, dist_f, wk1t, bk1, wk2t, bk2, wt1t, bt1, wt2t, bt2)


def _sc_scatter(vals_f, w2):
    T, K = vals_f.shape
    rows_per_w = T // _NW
    mesh = plsc.VectorSubcoreMesh(core_axis_name="c", subcore_axis_name="s")

    @functools.partial(
        pl.kernel,
        mesh=mesh,
        out_type=jax.ShapeDtypeStruct((T, _V), jnp.float32),
        scratch_types=[
            pltpu.VMEM((_V,), jnp.float32),
            pltpu.VMEM((K,), jnp.int32),
            pltpu.VMEM((K,), jnp.float32),
        ],
        compiler_params=pltpu.CompilerParams(needs_layout_passes=False),
    )
    def k(vals_hbm, w_hbm, out_hbm, row_v, idx_v, wv_v):
        cid = lax.axis_index("c")
        sid = lax.axis_index("s")
        wid = sid * 2 + cid

        def zero_chunk(i, carry):
            base = i * (10 * _L)
            for u in range(10):
                row_v[pl.ds(base + u * _L, _L)] = jnp.zeros((_L,), jnp.float32)
            return carry

        lax.fori_loop(0, _V // (10 * _L), zero_chunk, 0)

        for j in range(rows_per_w):
            t = wid * rows_per_w + j
            pltpu.sync_copy(vals_hbm.at[t], idx_v)
            pltpu.sync_copy(w_hbm.at[t], wv_v)
            for h in range(K // _L):
                iv = idx_v[pl.ds(h * _L, _L)]
                wv = wv_v[pl.ds(h * _L, _L)]
                plsc.store_scatter(row_v, [iv], wv)
            pltpu.sync_copy(row_v, out_hbm.at[t])
            if j < rows_per_w - 1:
                for h in range(K // _L):
                    iv = idx_v[pl.ds(h * _L, _L)]
                    plsc.store_scatter(row_v, [iv], jnp.zeros((_L,), jnp.float32))

    return k(vals_f, w2)


def kernel(vals, distances, Wk1, bk1, Wk2, bk2, Wl1, bl1, Wl2, bl2,
           Wt1, bt1, Wt2, bt2):
    B, S, K = vals.shape
    T = B * S
    vals_f = vals.reshape(T, K)
    dist_f = distances.reshape(T, K)
    w2 = _dense_pallas(vals_f, dist_f,
                       Wk1.T, bk1.reshape(1, -1), Wk2.T, bk2.reshape(1, -1),
                       Wt1.T, bt1.reshape(1, -1), Wt2.T, bt2.reshape(1, -1))
    out = _sc_scatter(vals_f, w2)
    return out.reshape(B, S, _V)


# speedup vs baseline: 4.2770x; 2.4156x over previous
import functools
import jax
import jax.numpy as jnp
from jax import lax
from jax.experimental import pallas as pl
from jax.experimental.pallas import tpu as pltpu
from jax.experimental.pallas import tpu_sc as plsc

_V = 100000
_R = 6
_NW = 32
_L = 16


def _dense_body(vals_ref, dist_ref, wk1_ref, bk1_ref, wk2_ref, bk2_ref,
                wt1_ref, bt1_ref, wt2_ref, bt2_ref, w2_ref):
    vals = vals_ref[...]
    dist = dist_ref[...]
    T, K = vals.shape

    eq = vals[:, :, None] == vals[:, None, :]
    ji = lax.broadcasted_iota(jnp.int32, (K, K), 0)
    mi = lax.broadcasted_iota(jnp.int32, (K, K), 1)
    seen = jnp.any(eq & (mi < ji)[None], axis=-1)
    isnew = ((vals != 0) & ~seen).astype(jnp.float32)
    cumtri = (mi <= ji).astype(jnp.float32)
    counts = lax.dot_general(isnew, cumtri, (((1,), (1,)), ((), ())),
                             preferred_element_type=jnp.float32)

    net_in = jnp.concatenate([dist, counts], axis=-1)

    h_k = jnp.tanh(jnp.dot(net_in, wk1_ref[...],
                           preferred_element_type=jnp.float32) + bk1_ref[...])
    logits = jnp.dot(h_k, wk2_ref[...],
                     preferred_element_type=jnp.float32) + bk2_ref[...]
    logits = logits - jnp.max(logits, axis=-1, keepdims=True)
    e = jnp.exp(logits)
    k_probs = e / jnp.sum(e, axis=-1, keepdims=True)

    h_t = jnp.tanh(jnp.dot(net_in, wt1_ref[...],
                           preferred_element_type=jnp.float32) + bt1_ref[...])
    tz = jnp.dot(h_t, wt2_ref[...],
                 preferred_element_type=jnp.float32) + bt2_ref[...]
    temp = 1.0 / (1.0 + jnp.exp(-tz))

    ri = lax.broadcasted_iota(jnp.int32, (_R, K), 0)
    ki = lax.broadcasted_iota(jnp.int32, (_R, K), 1)
    kmask = jnp.where(ki < (1 << ri), 1.0, 1000.0).astype(jnp.float32)

    d = -(dist[:, None, :] * kmask[None, :, :]) / temp[:, :, None]
    d = d - jnp.max(d, axis=-1, keepdims=True)
    ed = jnp.exp(d)
    kw = ed / jnp.sum(ed, axis=-1, keepdims=True)
    w = jnp.sum(k_probs[:, :, None] * kw, axis=1)

    w2_ref[...] = jnp.sum(eq.astype(jnp.float32) * w[:, None, :], axis=-1)


def _dense_pallas(vals_f, dist_f, wk1t, bk1, wk2t, bk2, wt1t, bt1, wt2t, bt2):
    T, K = vals_f.shape
    return pl.pallas_call(
        _dense_body,
        out_shape=jax.ShapeDtypeStruct((T, K), jnp.float32),
    )(vals_f, dist_f, wk1t, bk1, wk2t, bk2, wt1t, bt1, wt2t, bt2)


def _sc_scatter(vals_f, w2, B, S):
    T, K = vals_f.shape
    rows_per_w = T // _NW
    mesh = plsc.VectorSubcoreMesh(core_axis_name="c", subcore_axis_name="s")

    @functools.partial(
        pl.kernel,
        mesh=mesh,
        out_type=jax.ShapeDtypeStruct((B, S, _V), jnp.float32),
        scratch_types=[
            pltpu.VMEM((_V,), jnp.float32),
            pltpu.VMEM((K,), jnp.int32),
            pltpu.VMEM((K,), jnp.float32),
        ],
        compiler_params=pltpu.CompilerParams(needs_layout_passes=False),
    )
    def k(vals_hbm, w_hbm, out_hbm, row_v, idx_v, wv_v):
        cid = lax.axis_index("c")
        sid = lax.axis_index("s")
        wid = sid * 2 + cid

        def zero_chunk(i, carry):
            base = i * (10 * _L)
            for u in range(10):
                row_v[pl.ds(base + u * _L, _L)] = jnp.zeros((_L,), jnp.float32)
            return carry

        lax.fori_loop(0, _V // (10 * _L), zero_chunk, 0)

        for j in range(rows_per_w):
            t = wid * rows_per_w + j
            b = t // S
            s = t - b * S
            pltpu.sync_copy(vals_hbm.at[t], idx_v)
            pltpu.sync_copy(w_hbm.at[t], wv_v)
            for h in range(K // _L):
                iv = idx_v[pl.ds(h * _L, _L)]
                wv = wv_v[pl.ds(h * _L, _L)]
                plsc.store_scatter(row_v, [iv], wv)
            pltpu.sync_copy(row_v, out_hbm.at[b, s])
            if j < rows_per_w - 1:
                for h in range(K // _L):
                    iv = idx_v[pl.ds(h * _L, _L)]
                    plsc.store_scatter(row_v, [iv], jnp.zeros((_L,), jnp.float32))

    return k(vals_f, w2)


def kernel(vals, distances, Wk1, bk1, Wk2, bk2, Wl1, bl1, Wl2, bl2,
           Wt1, bt1, Wt2, bt2):
    B, S, K = vals.shape
    T = B * S
    vals_f = vals.reshape(T, K)
    dist_f = distances.reshape(T, K)
    w2 = _dense_pallas(vals_f, dist_f,
                       Wk1.T, bk1.reshape(1, -1), Wk2.T, bk2.reshape(1, -1),
                       Wt1.T, bt1.reshape(1, -1), Wt2.T, bt2.reshape(1, -1))
    return _sc_scatter(vals_f, w2, B, S)


# speedup vs baseline: 4.8003x; 1.1224x over previous
import functools
import jax
import jax.numpy as jnp
from jax import lax
from jax.experimental import pallas as pl
from jax.experimental.pallas import tpu as pltpu
from jax.experimental.pallas import tpu_sc as plsc

_V = 100000
_R = 6
_NW = 32
_L = 16


def _dense_body(vals_ref, dist_ref, wk1_ref, bk1_ref, wk2_ref, bk2_ref,
                wt1_ref, bt1_ref, wt2_ref, bt2_ref, w2_ref):
    vals = vals_ref[...]
    dist = dist_ref[...]
    T, K = vals.shape

    eq = vals[:, :, None] == vals[:, None, :]
    ji = lax.broadcasted_iota(jnp.int32, (K, K), 0)
    mi = lax.broadcasted_iota(jnp.int32, (K, K), 1)
    seen = jnp.any(eq & (mi < ji)[None], axis=-1)
    isnew = ((vals != 0) & ~seen).astype(jnp.float32)
    cumtri = (mi <= ji).astype(jnp.float32)
    counts = lax.dot_general(isnew, cumtri, (((1,), (1,)), ((), ())),
                             preferred_element_type=jnp.float32)

    net_in = jnp.concatenate([dist, counts], axis=-1)

    h_k = jnp.tanh(jnp.dot(net_in, wk1_ref[...].T,
                           preferred_element_type=jnp.float32) + bk1_ref[...])
    logits = jnp.dot(h_k, wk2_ref[...].T,
                     preferred_element_type=jnp.float32) + bk2_ref[...]
    logits = logits - jnp.max(logits, axis=-1, keepdims=True)
    e = jnp.exp(logits)
    k_probs = e / jnp.sum(e, axis=-1, keepdims=True)

    h_t = jnp.tanh(jnp.dot(net_in, wt1_ref[...].T,
                           preferred_element_type=jnp.float32) + bt1_ref[...])
    tz = jnp.sum(h_t * wt2_ref[...], axis=-1, keepdims=True) + bt2_ref[...]
    temp = 1.0 / (1.0 + jnp.exp(-tz))

    ri = lax.broadcasted_iota(jnp.int32, (_R, K), 0)
    ki = lax.broadcasted_iota(jnp.int32, (_R, K), 1)
    kmask = jnp.where(ki < (1 << ri), 1.0, 1000.0).astype(jnp.float32)

    d = -(dist[:, None, :] * kmask[None, :, :]) / temp[:, :, None]
    d = d - jnp.max(d, axis=-1, keepdims=True)
    ed = jnp.exp(d)
    kw = ed / jnp.sum(ed, axis=-1, keepdims=True)
    w = jnp.sum(k_probs[:, :, None] * kw, axis=1)

    w2_ref[...] = jnp.sum(eq.astype(jnp.float32) * w[:, None, :], axis=-1)


def _dense_pallas(vals_f, dist_f, wk1t, bk1, wk2t, bk2, wt1t, bt1, wt2t, bt2):
    T, K = vals_f.shape
    return pl.pallas_call(
        _dense_body,
        out_shape=jax.ShapeDtypeStruct((T, K), jnp.float32),
    )(vals_f, dist_f, wk1t, bk1, wk2t, bk2, wt1t, bt1, wt2t, bt2)


def _sc_scatter(vals_f, w2, B, S):
    T, K = vals_f.shape
    rows_per_w = T // _NW
    mesh = plsc.VectorSubcoreMesh(core_axis_name="c", subcore_axis_name="s")

    @functools.partial(
        pl.kernel,
        mesh=mesh,
        out_type=jax.ShapeDtypeStruct((B, S, _V), jnp.float32),
        scratch_types=[
            pltpu.VMEM((_V,), jnp.float32),
            pltpu.VMEM((K,), jnp.int32),
            pltpu.VMEM((K,), jnp.float32),
        ],
        compiler_params=pltpu.CompilerParams(needs_layout_passes=False),
    )
    def k(vals_hbm, w_hbm, out_hbm, row_v, idx_v, wv_v):
        cid = lax.axis_index("c")
        sid = lax.axis_index("s")
        wid = sid * 2 + cid

        def zero_chunk(i, carry):
            base = i * (10 * _L)
            for u in range(10):
                row_v[pl.ds(base + u * _L, _L)] = jnp.zeros((_L,), jnp.float32)
            return carry

        lax.fori_loop(0, _V // (10 * _L), zero_chunk, 0)

        for j in range(rows_per_w):
            t = wid * rows_per_w + j
            b = t // S
            s = t - b * S
            pltpu.sync_copy(vals_hbm.at[t], idx_v)
            pltpu.sync_copy(w_hbm.at[t], wv_v)
            for h in range(K // _L):
                iv = idx_v[pl.ds(h * _L, _L)]
                wv = wv_v[pl.ds(h * _L, _L)]
                plsc.store_scatter(row_v, [iv], wv)
            pltpu.sync_copy(row_v, out_hbm.at[b, s])
            if j < rows_per_w - 1:
                for h in range(K // _L):
                    iv = idx_v[pl.ds(h * _L, _L)]
                    plsc.store_scatter(row_v, [iv], jnp.zeros((_L,), jnp.float32))

    return k(vals_f, w2)


def kernel(vals, distances, Wk1, bk1, Wk2, bk2, Wl1, bl1, Wl2, bl2,
           Wt1, bt1, Wt2, bt2):
    B, S, K = vals.shape
    T = B * S
    vals_f = vals.reshape(T, K)
    dist_f = distances.reshape(T, K)
    w2 = _dense_pallas(vals_f, dist_f,
                       Wk1, bk1.reshape(1, -1), Wk2, bk2.reshape(1, -1),
                       Wt1, bt1.reshape(1, -1), Wt2, bt2.reshape(1, -1))
    return _sc_scatter(vals_f, w2, B, S)


# speedup vs baseline: 4.8046x; 1.0009x over previous
import functools
import jax
import jax.numpy as jnp
from jax import lax
from jax.experimental import pallas as pl
from jax.experimental.pallas import tpu as pltpu
from jax.experimental.pallas import tpu_sc as plsc

_V = 100000
_R = 6
_NW = 32
_L = 16


def _dense_body(vals_ref, dist_ref, wk1_ref, bk1_ref, wk2_ref, bk2_ref,
                wt1_ref, bt1_ref, wt2_ref, bt2_ref, w2_ref):
    vals = vals_ref[...]
    dist = dist_ref[...]
    T, K = vals.shape

    eq = vals[:, :, None] == vals[:, None, :]
    eq_f = eq.astype(jnp.float32)
    ji = lax.broadcasted_iota(jnp.int32, (K, K), 0)
    mi = lax.broadcasted_iota(jnp.int32, (K, K), 1)
    lower = (mi < ji).astype(jnp.float32)
    ones_col = jnp.ones((K, 1), jnp.float32)
    seen_sum = jnp.dot((eq_f * lower[None]).reshape(T * K, K), ones_col,
                       preferred_element_type=jnp.float32).reshape(T, K)
    isnew = jnp.where((vals != 0) & (seen_sum < 0.5), 1.0, 0.0)
    cumtri = (mi <= ji).astype(jnp.float32)
    counts = lax.dot_general(isnew, cumtri, (((1,), (1,)), ((), ())),
                             preferred_element_type=jnp.float32)

    net_in = jnp.concatenate([dist, counts], axis=-1)

    h_k = jnp.tanh(jnp.dot(net_in, wk1_ref[...].T,
                           preferred_element_type=jnp.float32) + bk1_ref[...])
    logits = jnp.dot(h_k, wk2_ref[...].T,
                     preferred_element_type=jnp.float32) + bk2_ref[...]
    logits = logits - jnp.max(logits, axis=-1, keepdims=True)
    e = jnp.exp(logits)
    k_probs = e / jnp.sum(e, axis=-1, keepdims=True)

    h_t = jnp.tanh(jnp.dot(net_in, wt1_ref[...].T,
                           preferred_element_type=jnp.float32) + bt1_ref[...])
    tz = jnp.sum(h_t * wt2_ref[...], axis=-1, keepdims=True) + bt2_ref[...]
    temp = 1.0 / (1.0 + jnp.exp(-tz))

    ri = lax.broadcasted_iota(jnp.int32, (_R, K), 0)
    ki = lax.broadcasted_iota(jnp.int32, (_R, K), 1)
    kmask = jnp.where(ki < (1 << ri), 1.0, 1000.0).astype(jnp.float32)

    d = -(dist[:, None, :] * kmask[None, :, :]) / temp[:, :, None]
    d = d - jnp.max(d, axis=-1, keepdims=True)
    ed = jnp.exp(d)
    kw = ed / jnp.sum(ed, axis=-1, keepdims=True)
    w = jnp.sum(k_probs[:, :, None] * kw, axis=1)

    w2_ref[...] = jnp.dot((eq_f * w[:, None, :]).reshape(T * K, K), ones_col,
                          preferred_element_type=jnp.float32).reshape(T, K)


def _dense_pallas(vals_f, dist_f, wk1t, bk1, wk2t, bk2, wt1t, bt1, wt2t, bt2):
    T, K = vals_f.shape
    return pl.pallas_call(
        _dense_body,
        out_shape=jax.ShapeDtypeStruct((T, K), jnp.float32),
    )(vals_f, dist_f, wk1t, bk1, wk2t, bk2, wt1t, bt1, wt2t, bt2)


def _sc_scatter(vals_f, w2, B, S):
    T, K = vals_f.shape
    rows_per_w = T // _NW
    mesh = plsc.VectorSubcoreMesh(core_axis_name="c", subcore_axis_name="s")

    @functools.partial(
        pl.kernel,
        mesh=mesh,
        out_type=jax.ShapeDtypeStruct((B, S, _V), jnp.float32),
        scratch_types=[
            pltpu.VMEM((_V,), jnp.float32),
            pltpu.VMEM((K,), jnp.int32),
            pltpu.VMEM((K,), jnp.float32),
        ],
        compiler_params=pltpu.CompilerParams(needs_layout_passes=False),
    )
    def k(vals_hbm, w_hbm, out_hbm, row_v, idx_v, wv_v):
        cid = lax.axis_index("c")
        sid = lax.axis_index("s")
        wid = sid * 2 + cid

        def zero_chunk(i, carry):
            base = i * (25 * _L)
            for u in range(25):
                row_v[pl.ds(base + u * _L, _L)] = jnp.zeros((_L,), jnp.float32)
            return carry

        lax.fori_loop(0, _V // (25 * _L), zero_chunk, 0)

        for j in range(rows_per_w):
            t = wid * rows_per_w + j
            b = t // S
            s = t - b * S
            pltpu.sync_copy(vals_hbm.at[t], idx_v)
            pltpu.sync_copy(w_hbm.at[t], wv_v)
            for h in range(K // _L):
                iv = idx_v[pl.ds(h * _L, _L)]
                wv = wv_v[pl.ds(h * _L, _L)]
                plsc.store_scatter(row_v, [iv], wv)
            pltpu.sync_copy(row_v, out_hbm.at[b, s])
            if j < rows_per_w - 1:
                for h in range(K // _L):
                    iv = idx_v[pl.ds(h * _L, _L)]
                    plsc.store_scatter(row_v, [iv], jnp.zeros((_L,), jnp.float32))

    return k(vals_f, w2)


def kernel(vals, distances, Wk1, bk1, Wk2, bk2, Wl1, bl1, Wl2, bl2,
           Wt1, bt1, Wt2, bt2):
    B, S, K = vals.shape
    T = B * S
    vals_f = vals.reshape(T, K)
    dist_f = distances.reshape(T, K)
    w2 = _dense_pallas(vals_f, dist_f,
                       Wk1, bk1.reshape(1, -1), Wk2, bk2.reshape(1, -1),
                       Wt1, bt1.reshape(1, -1), Wt2, bt2.reshape(1, -1))
    return _sc_scatter(vals_f, w2, B, S)


# speedup vs baseline: 4.8106x; 1.0013x over previous
import functools
import jax
import jax.numpy as jnp
from jax import lax
from jax.experimental import pallas as pl
from jax.experimental.pallas import tpu as pltpu
from jax.experimental.pallas import tpu_sc as plsc

_V = 100000
_R = 6
_NW = 32
_L = 16


def _dense_body(vals_ref, dist_ref, wk1_ref, bk1_ref, wk2_ref, bk2_ref,
                wt1_ref, bt1_ref, wt2_ref, bt2_ref, w2_ref):
    vals = vals_ref[...]
    dist = dist_ref[...]
    T, K = vals.shape

    eq = vals[:, :, None] == vals[:, None, :]
    eq_f = eq.astype(jnp.float32)
    ji = lax.broadcasted_iota(jnp.int32, (K, K), 0)
    mi = lax.broadcasted_iota(jnp.int32, (K, K), 1)
    lower = (mi < ji).astype(jnp.float32)
    ones_col = jnp.ones((K, 1), jnp.float32)
    seen_sum = jnp.dot((eq_f * lower[None]).reshape(T * K, K), ones_col,
                       preferred_element_type=jnp.float32).reshape(T, K)
    isnew = jnp.where((vals != 0) & (seen_sum < 0.5), 1.0, 0.0)
    cumtri = (mi <= ji).astype(jnp.float32)
    counts = lax.dot_general(isnew, cumtri, (((1,), (1,)), ((), ())),
                             preferred_element_type=jnp.float32)

    net_in = jnp.concatenate([dist, counts], axis=-1)

    h_k = jnp.tanh(jnp.dot(net_in, wk1_ref[...].T,
                           preferred_element_type=jnp.float32) + bk1_ref[...])
    logits = jnp.dot(h_k, wk2_ref[...].T,
                     preferred_element_type=jnp.float32) + bk2_ref[...]
    logits = logits - jnp.max(logits, axis=-1, keepdims=True)
    e = jnp.exp(logits)
    k_probs = e / jnp.sum(e, axis=-1, keepdims=True)

    h_t = jnp.tanh(jnp.dot(net_in, wt1_ref[...].T,
                           preferred_element_type=jnp.float32) + bt1_ref[...])
    tz = jnp.sum(h_t * wt2_ref[...], axis=-1, keepdims=True) + bt2_ref[...]
    temp = 1.0 / (1.0 + jnp.exp(-tz))

    ri = lax.broadcasted_iota(jnp.int32, (_R, K), 0)
    ki = lax.broadcasted_iota(jnp.int32, (_R, K), 1)
    kmask = jnp.where(ki < (1 << ri), 1.0, 1000.0).astype(jnp.float32)

    d = -(dist[:, None, :] * kmask[None, :, :]) / temp[:, :, None]
    d = d - jnp.max(d, axis=-1, keepdims=True)
    ed = jnp.exp(d)
    kw = ed / jnp.sum(ed, axis=-1, keepdims=True)
    w = jnp.sum(k_probs[:, :, None] * kw, axis=1)

    w2_ref[...] = jnp.dot((eq_f * w[:, None, :]).reshape(T * K, K), ones_col,
                          preferred_element_type=jnp.float32).reshape(T, K)


def _dense_pallas(vals_f, dist_f, wk1t, bk1, wk2t, bk2, wt1t, bt1, wt2t, bt2):
    T, K = vals_f.shape
    return pl.pallas_call(
        _dense_body,
        out_shape=jax.ShapeDtypeStruct((T, K), jnp.float32),
    )(vals_f, dist_f, wk1t, bk1, wk2t, bk2, wt1t, bt1, wt2t, bt2)


def _sc_scatter(vals_f, w2, B, S):
    T, K = vals_f.shape
    rows_per_w = T // _NW
    mesh = plsc.VectorSubcoreMesh(core_axis_name="c", subcore_axis_name="s")

    H = 49920
    H2 = _V - H

    @functools.partial(
        pl.kernel,
        mesh=mesh,
        out_type=jax.ShapeDtypeStruct((B, S, _V), jnp.float32),
        scratch_types=[
            pltpu.VMEM((H,), jnp.float32),
            pltpu.VMEM((H2,), jnp.float32),
            pltpu.VMEM((K,), jnp.int32),
            pltpu.VMEM((K,), jnp.float32),
            pltpu.SemaphoreType.DMA,
        ],
        compiler_params=pltpu.CompilerParams(needs_layout_passes=False),
    )
    def k(vals_hbm, w_hbm, out_hbm, rowa_v, rowb_v, idx_v, wv_v, sem):
        cid = lax.axis_index("c")
        sid = lax.axis_index("s")
        wid = sid * 2 + cid

        def zero_a(i, carry):
            base = i * (10 * _L)
            for u in range(10):
                rowa_v[pl.ds(base + u * _L, _L)] = jnp.zeros((_L,), jnp.float32)
            return carry

        def zero_b(i, carry):
            base = i * (10 * _L)
            for u in range(10):
                rowb_v[pl.ds(base + u * _L, _L)] = jnp.zeros((_L,), jnp.float32)
            return carry

        lax.fori_loop(0, H // (10 * _L), zero_a, 0)
        lax.fori_loop(0, H2 // (10 * _L), zero_b, 0)

        def scatter_halves(x16, iv):
            ma = iv < H
            plsc.store_scatter(rowa_v, [jnp.where(ma, iv, 0)], x16, mask=ma)
            plsc.store_scatter(rowb_v, [jnp.where(ma, 0, iv - H)], x16,
                               mask=~ma)

        for j in range(rows_per_w):
            t = wid * rows_per_w + j
            b = t // S
            s = t - b * S
            pltpu.sync_copy(vals_hbm.at[t], idx_v)
            pltpu.sync_copy(w_hbm.at[t], wv_v)
            for h in range(K // _L):
                scatter_halves(wv_v[pl.ds(h * _L, _L)],
                               idx_v[pl.ds(h * _L, _L)])
            ca = pltpu.async_copy(rowa_v, out_hbm.at[b, s, pl.ds(0, H)], sem)
            cb = pltpu.async_copy(rowb_v, out_hbm.at[b, s, pl.ds(H, H2)], sem)
            ca.wait()
            cb.wait()
            if j < rows_per_w - 1:
                for h in range(K // _L):
                    scatter_halves(jnp.zeros((_L,), jnp.float32),
                                   idx_v[pl.ds(h * _L, _L)])

    return k(vals_f, w2)


def kernel(vals, distances, Wk1, bk1, Wk2, bk2, Wl1, bl1, Wl2, bl2,
           Wt1, bt1, Wt2, bt2):
    B, S, K = vals.shape
    T = B * S
    vals_f = vals.reshape(T, K)
    dist_f = distances.reshape(T, K)
    w2 = _dense_pallas(vals_f, dist_f,
                       Wk1, bk1.reshape(1, -1), Wk2, bk2.reshape(1, -1),
                       Wt1, bt1.reshape(1, -1), Wt2, bt2.reshape(1, -1))
    return _sc_scatter(vals_f, w2, B, S)
